# baseline (speedup 1.0000x reference)
import functools

import jax
import jax.numpy as jnp
from jax import lax
from jax.experimental import pallas as pl
from jax.experimental.pallas import tpu as pltpu
from jax.experimental.pallas import tpu_sc as plsc

BATCH = 4096
SEQ = 200
EMBED = 64
VOCAB = 1000000
HALF = SEQ // 2

NUM_CORES = 2
NUM_SUBCORES = 16
NUM_WORKERS = NUM_CORES * NUM_SUBCORES
BPW = BATCH // NUM_WORKERS
G = 4
NGRP = BPW // G

_CHUNKS = (0, 16, 32, 48, 64, 80, 84)


@functools.partial(
    pl.kernel,
    out_type=jax.ShapeDtypeStruct((BATCH, 2, HALF, EMBED), jnp.float32),
    mesh=plsc.VectorSubcoreMesh(core_axis_name="c", subcore_axis_name="s"),
    compiler_params=pltpu.CompilerParams(
        use_tc_tiling_on_sc=False, needs_layout_passes=False),
    scratch_types=[
        pltpu.VMEM((G, 2, HALF), jnp.int32),
        pltpu.VMEM((2, G, 2, HALF), jnp.int32),
        pltpu.VMEM((2, G, 2, HALF, EMBED), jnp.float32),
        pltpu.VMEM((2, HALF, EMBED), jnp.float32),
        pltpu.SemaphoreType.DMA,
        pltpu.SemaphoreType.DMA,
        pltpu.SemaphoreType.DMA,
        pltpu.SemaphoreType.DMA,
    ],
)
def _sc_embed(x_hbm, tok_hbm, pos_hbm, out_hbm,
              raw_v, idx_v, rows_v, pos_v, g0, g1, o0, o1):
    wid = lax.axis_index("s") * NUM_CORES + lax.axis_index("c")
    base = wid * BPW

    pltpu.sync_copy(pos_hbm, pos_v)

    def stage_and_fire(j, p):
        b0 = base + j * G
        pltpu.sync_copy(x_hbm.at[pl.ds(b0, G)], raw_v)
        for g in range(G):
            for h in range(2):
                for c in _CHUNKS:
                    v = raw_v[g, h, pl.ds(c, 16)]
                    idx_v[p, g, h, pl.ds(c, 16)] = v + v
        gsem = (g0, g1)[p]
        for g in range(G):
            for h in range(2):
                pltpu.async_copy(
                    tok_hbm.at[idx_v.at[p, g, h]], rows_v.at[p, g, h], gsem)

    def wait_gathers(p):
        gsem = (g0, g1)[p]
        for _ in range(G * 2):
            pltpu.make_async_copy(
                tok_hbm.at[idx_v.at[0, 0, 0]], rows_v.at[p, 0, 0], gsem
            ).wait()

    def store(j, p):
        osem = (o0, o1)[p]
        return pltpu.async_copy(
            rows_v.at[p], out_hbm.at[pl.ds(base + j * G, G)], osem)

    def wait_store(j, p):
        osem = (o0, o1)[p]
        pltpu.make_async_copy(
            rows_v.at[p], out_hbm.at[pl.ds(base + j * G, G)], osem).wait()

    stage_and_fire(0, 0)

    def j_body(step, carry):
        for p in range(2):
            j = step * 2 + p

            wait_gathers(p)

            @pl.when(j + 1 < NGRP)
            def _():
                @pl.when(j >= 1)
                def _():
                    wait_store(j - 1, 1 - p)
                stage_and_fire(j + 1, 1 - p)

            def r_body(r, rcarry):
                for h in range(2):
                    prow = [pos_v[h, r, pl.ds(16 * c, 16)] for c in range(4)]
                    for g in range(G):
                        for c in range(4):
                            sl = pl.ds(16 * c, 16)
                            rows_v[p, g, h, r, sl] = (
                                rows_v[p, g, h, r, sl] + prow[c])
                return rcarry
            lax.fori_loop(0, HALF, r_body, 0, unroll=2)

            store(j, p)
        return carry

    lax.fori_loop(0, NGRP // 2, j_body, 0)
    wait_store(NGRP - 2, 0)
    wait_store(NGRP - 1, 1)


def kernel(x, token_table, position_table):
    x2 = x.astype(jnp.int32).reshape(BATCH, 2, HALF)
    tp = jnp.pad(token_table, ((0, 0), (0, 128 - EMBED))).reshape(2 * VOCAB, EMBED)
    pos2 = position_table.reshape(2, HALF, EMBED)
    out = _sc_embed(x2, tp, pos2)
    return out.reshape(BATCH, SEQ, EMBED)
